# Initial kernel scaffold; baseline (speedup 1.0000x reference)
import jax
import jax.numpy as jnp
from jax.experimental import pallas as pl


def kernel(head, ent_emb):
    raise NotImplementedError("write your pallas kernel here")


# speedup vs baseline: 1.3340x; 1.3340x over previous
import functools

import jax
import jax.numpy as jnp
from jax.experimental import pallas as pl

_EMB = 16
_TN = 2048


def _normalize_rows(x):
    n = jnp.sqrt(jnp.sum(x * x, axis=-1, keepdims=True))
    return x / jnp.maximum(n, 1e-12)


def _score_kernel(head_ref, ent_ref, out_ref):
    head = head_ref[...]
    b = head.shape[0] // 2
    h = _normalize_rows(head[:b])
    r = _normalize_rows(head[b:])
    re_h, im_h = h[:, :_EMB], h[:, _EMB:]
    re_r, im_r = r[:, :_EMB], r[:, _EMB:]
    p = jnp.concatenate(
        [re_h * re_r - im_h * im_r, re_h * im_r + im_h * re_r], axis=1
    )
    t = _normalize_rows(ent_ref[...])
    score = jax.lax.dot_general(
        p, t, (((1,), (1,)), ((), ())), preferred_element_type=jnp.float32
    )
    out_ref[...] = jax.nn.sigmoid(score)


def kernel(head, ent_emb):
    two_b, width = head.shape
    batch = two_b // 2
    ent_num = ent_emb.shape[0]
    grid = pl.cdiv(ent_num, _TN)
    return pl.pallas_call(
        _score_kernel,
        grid=(grid,),
        in_specs=[
            pl.BlockSpec((two_b, width), lambda i: (0, 0)),
            pl.BlockSpec((_TN, width), lambda i: (i, 0)),
        ],
        out_specs=pl.BlockSpec((batch, _TN), lambda i: (0, i)),
        out_shape=jax.ShapeDtypeStruct((batch, ent_num), jnp.float32),
    )(head, ent_emb)


# speedup vs baseline: 1.6091x; 1.2062x over previous
import functools

import jax
import jax.numpy as jnp
from jax.experimental import pallas as pl

_EMB = 16
_TN = 4096


def _normalize_rows(x):
    n = jnp.sqrt(jnp.sum(x * x, axis=-1, keepdims=True))
    return x / jnp.maximum(n, 1e-12)


def _score_kernel(head_ref, ent_ref, out_ref):
    head = head_ref[...]
    b = head.shape[0] // 2
    h = _normalize_rows(head[:b])
    r = _normalize_rows(head[b:])
    re_h, im_h = h[:, :_EMB], h[:, _EMB:]
    re_r, im_r = r[:, :_EMB], r[:, _EMB:]
    p = jnp.concatenate(
        [re_h * re_r - im_h * im_r, re_h * im_r + im_h * re_r], axis=1
    )
    t = _normalize_rows(ent_ref[...])
    score = jax.lax.dot_general(
        p, t, (((1,), (1,)), ((), ())), preferred_element_type=jnp.float32
    )
    out_ref[...] = 0.5 * jnp.tanh(0.5 * score) + 0.5


def kernel(head, ent_emb):
    two_b, width = head.shape
    batch = two_b // 2
    ent_num = ent_emb.shape[0]
    grid = pl.cdiv(ent_num, _TN)
    return pl.pallas_call(
        _score_kernel,
        grid=(grid,),
        in_specs=[
            pl.BlockSpec((two_b, width), lambda i: (0, 0)),
            pl.BlockSpec((_TN, width), lambda i: (i, 0)),
        ],
        out_specs=pl.BlockSpec((batch, _TN), lambda i: (0, i)),
        out_shape=jax.ShapeDtypeStruct((batch, ent_num), jnp.float32),
    )(head, ent_emb)
